# Initial kernel scaffold; baseline (speedup 1.0000x reference)
import jax
import jax.numpy as jnp
from jax.experimental import pallas as pl


def kernel(x, edge_index, edge_attr, W0, b0, W1, b1, W2, b2):
    raise NotImplementedError("write your pallas kernel here")


# speedup vs baseline: 8.1298x; 8.1298x over previous
import functools

import jax
import jax.numpy as jnp
from jax import lax
from jax.experimental import pallas as pl
from jax.experimental.pallas import tpu as pltpu
from jax.experimental.pallas import tpu_sc as plsc

N_NODES = 10000
N_EDGES = 320000
D_HID = 128
D_OUT_PAD = 64

NC = 2
NS = 16
NW = NC * NS
EPW = N_EDGES // NW
EK = 80
N_PAD = 10240
ROWS_PER_SUB = N_PAD // NS

_sc_mesh = functools.partial(
    plsc.VectorSubcoreMesh, core_axis_name="c", subcore_axis_name="s")


@functools.partial(
    pl.kernel,
    mesh=_sc_mesh(),
    out_type=jax.ShapeDtypeStruct((NC * N_NODES,), jnp.float32),
    scratch_types=[
        pltpu.VMEM((EK,), jnp.int32),
        pltpu.VMEM((EK,), jnp.float32),
        pltpu.VMEM((N_NODES,), jnp.float32),
        pltpu.VMEM_SHARED((N_NODES,), jnp.float32),
    ],
)
def _deg_kernel(dst_hbm, ew_hbm, zeros_hbm, out_hbm, dst_v, w_v, deg_v, acc_sh):
    c = lax.axis_index("c")
    s = lax.axis_index("s")
    wid = s * NC + c

    @pl.when(s == 0)
    def _():
        pltpu.sync_copy(zeros_hbm, deg_v)
        pltpu.sync_copy(deg_v, acc_sh)

    plsc.subcore_barrier()

    def chunk(i, carry):
        base = wid * EPW + i * EK
        pltpu.sync_copy(dst_hbm.at[pl.ds(base, EK)], dst_v)
        pltpu.sync_copy(ew_hbm.at[pl.ds(base, EK)], w_v)
        pltpu.sync_copy(w_v, acc_sh.at[dst_v], add=True)
        return carry

    lax.fori_loop(0, EPW // EK, chunk, 0)
    plsc.subcore_barrier()

    @pl.when(s == 0)
    def _():
        pltpu.sync_copy(acc_sh, deg_v)
        pltpu.sync_copy(deg_v, out_hbm.at[pl.ds(c * N_NODES, N_NODES)])


def _make_agg_kernel(d):
    nv = d // 16

    @functools.partial(
        pl.kernel,
        mesh=_sc_mesh(),
        out_type=jax.ShapeDtypeStruct((NC, N_PAD, d), jnp.float32),
        scratch_types=[
            pltpu.VMEM((EK,), jnp.int32),
            pltpu.VMEM((EK,), jnp.int32),
            pltpu.VMEM((EK,), jnp.float32),
            pltpu.VMEM((EK, d), jnp.float32),
            pltpu.VMEM_SHARED((N_PAD, d), jnp.float32),
            pltpu.SemaphoreType.DMA,
        ],
    )
    def agg(g_hbm, src_hbm, dst_hbm, ew_hbm, zeros_hbm, out_hbm,
            src_v, dst_v, w_v, rows_v, acc_sh, sem):
        c = lax.axis_index("c")
        s = lax.axis_index("s")
        wid = s * NC + c
        n_slab = ROWS_PER_SUB // EK
        pltpu.sync_copy(zeros_hbm, rows_v)
        for t in range(n_slab):
            pltpu.sync_copy(
                rows_v, acc_sh.at[pl.ds(s * ROWS_PER_SUB + t * EK, EK)])
        plsc.subcore_barrier()

        def chunk(i, carry):
            base = wid * EPW + i * EK
            pltpu.sync_copy(src_hbm.at[pl.ds(base, EK)], src_v)
            pltpu.sync_copy(dst_hbm.at[pl.ds(base, EK)], dst_v)
            pltpu.sync_copy(ew_hbm.at[pl.ds(base, EK)], w_v)
            pltpu.async_copy(g_hbm.at[src_v], rows_v, sem).wait()

            def scale(g, cc):
                wv = w_v[pl.ds(g * 16, 16)]
                for k in range(16):
                    e = g * 16 + k
                    w = wv[k]
                    for j in range(nv):
                        sl = pl.ds(j * 16, 16)
                        rows_v[e, sl] = rows_v[e, sl] * w
                return cc

            lax.fori_loop(0, EK // 16, scale, 0)
            pltpu.sync_copy(rows_v, acc_sh.at[dst_v], add=True)
            return carry

        lax.fori_loop(0, EPW // EK, chunk, 0)
        plsc.subcore_barrier()
        for t in range(n_slab):
            rs = pl.ds(s * ROWS_PER_SUB + t * EK, EK)
            pltpu.sync_copy(acc_sh.at[rs], rows_v)
            pltpu.sync_copy(rows_v, out_hbm.at[c, rs])

    return agg


_agg128 = _make_agg_kernel(D_HID)


_ROWS = 400
_GRID = N_NODES // _ROWS


def _k0_body(x_ref, w_ref, degp_ref, dinv_ref, g_ref):
    deg = degp_ref[0] + degp_ref[1] + 1.0
    dinv = jnp.where(deg > 0, lax.rsqrt(deg), 0.0)
    dinv_ref[...] = dinv
    g_ref[...] = dinv * jnp.dot(x_ref[...], w_ref[...],
                                preferred_element_type=jnp.float32)


def _kmid_body(p_ref, gp_ref, dinv_ref, b_ref, w_ref, g_ref, *, relu):
    dinv = dinv_ref[...]
    h = dinv * (p_ref[0] + p_ref[1] + gp_ref[...]) + b_ref[...]
    if relu:
        h = jnp.maximum(h, 0.0)
    g_ref[...] = dinv * jnp.dot(h, w_ref[...],
                                preferred_element_type=jnp.float32)


def _kelem_body(p_ref, gp_ref, dinv_ref, b_ref, u_ref):
    dinv = dinv_ref[...]
    u_ref[...] = dinv * (dinv * (p_ref[0] + p_ref[1] + gp_ref[...])
                         + b_ref[...])


def _kfin_body(p_ref, u_ref, dinv_ref, w_ref, b_ref, o_ref):
    o_ref[...] = dinv_ref[...] * jnp.dot(
        p_ref[0] + p_ref[1] + u_ref[...], w_ref[...],
        preferred_element_type=jnp.float32) + b_ref[...]


def _row_spec(d):
    return pl.BlockSpec((_ROWS, d), lambda i: (i, 0))


def _part_spec(d):
    return pl.BlockSpec((NC, _ROWS, d), lambda i: (0, i, 0))


def _full_spec(r, c):
    return pl.BlockSpec((r, c), lambda i: (0, 0))


def _k0(x, w0, degp):
    return pl.pallas_call(
        _k0_body,
        grid=(_GRID,),
        in_specs=[_row_spec(D_HID), _full_spec(D_HID, D_HID), _part_spec(1)],
        out_specs=[_row_spec(1), _row_spec(D_HID)],
        out_shape=[jax.ShapeDtypeStruct((N_NODES, 1), jnp.float32),
                   jax.ShapeDtypeStruct((N_NODES, D_HID), jnp.float32)],
    )(x, w0, degp)


def _kmid(p, gp, dinv, b, w, relu):
    return pl.pallas_call(
        functools.partial(_kmid_body, relu=relu),
        grid=(_GRID,),
        in_specs=[_part_spec(D_HID), _row_spec(D_HID), _row_spec(1),
                  _full_spec(1, D_HID), _full_spec(D_HID, w.shape[1])],
        out_specs=_row_spec(w.shape[1]),
        out_shape=jax.ShapeDtypeStruct((N_NODES, w.shape[1]), jnp.float32),
    )(p, gp, dinv, b, w)


def _kelem(p, gp, dinv, b):
    return pl.pallas_call(
        _kelem_body,
        grid=(_GRID,),
        in_specs=[_part_spec(D_HID), _row_spec(D_HID), _row_spec(1),
                  _full_spec(1, D_HID)],
        out_specs=_row_spec(D_HID),
        out_shape=jax.ShapeDtypeStruct((N_NODES, D_HID), jnp.float32),
    )(p, gp, dinv, b)


def _kfin(p, u, dinv, w2, b2):
    ncls = w2.shape[1]
    return pl.pallas_call(
        _kfin_body,
        grid=(_GRID,),
        in_specs=[_part_spec(D_HID), _row_spec(D_HID), _row_spec(1),
                  _full_spec(D_HID, ncls), _full_spec(1, ncls)],
        out_specs=_row_spec(ncls),
        out_shape=jax.ShapeDtypeStruct((N_NODES, ncls), jnp.float32),
    )(p, u, dinv, w2, b2)


def kernel(x, edge_index, edge_attr, W0, b0, W1, b1, W2, b2):
    src = edge_index[0]
    dst = edge_index[1]
    ew = edge_attr

    zeros_n = jnp.zeros((N_NODES,), jnp.float32)
    zeros128 = jnp.zeros((EK, D_HID), jnp.float32)

    degp = _deg_kernel(dst, ew, zeros_n).reshape(NC, N_NODES, 1)

    dinv, g0 = _k0(x, W0, degp)
    p0 = _agg128(g0, src, dst, ew, zeros128)
    g1 = _kmid(p0, g0, dinv, b0.reshape(1, D_HID), W1, relu=True)
    p1 = _agg128(g1, src, dst, ew, zeros128)
    u2 = _kelem(p1, g1, dinv, b1.reshape(1, D_HID))
    p2 = _agg128(u2, src, dst, ew, zeros128)
    return _kfin(p2, u2, dinv, W2, b2.reshape(1, W2.shape[1]))
